# Initial kernel scaffold; baseline (speedup 1.0000x reference)
import jax
import jax.numpy as jnp
from jax.experimental import pallas as pl


def kernel(x):
    raise NotImplementedError("write your pallas kernel here")


# speedup vs baseline: 3.2220x; 3.2220x over previous
import functools

import jax
import jax.numpy as jnp
import numpy as np
from jax import lax
from jax.experimental import pallas as pl
from jax.experimental.pallas import tpu as pltpu
from jax.experimental.pallas import tpu_sc as plsc

B = 128
N = 32768
K = 64
L = 16
NC, NS = 2, 16
NW = NC * NS
ROWS_PER_W = B // NW
NV = N // L

_I32_MIN = np.int32(-2147483648)
_M7F = np.int32(0x7FFFFFFF)


def _mono_key(v):
    b = lax.bitcast_convert_type(v, jnp.int32)
    f = b >> 31
    return b ^ (f | _I32_MIN)


def _as_u(k):
    return lax.bitcast_convert_type(k, jnp.uint32)


def _search(hist, iota16, TOT, r):
    needP = TOT - r + 1

    def body(j, c):
        found, dstar, pstar, cstar, prefix = c
        t = jnp.zeros((L,), jnp.int32)
        z = jnp.zeros((L,), jnp.int32)
        for lane in range(L):
            off = lane * 256 + j * L
            t = t + hist[pl.ds(off, L)]
            hist[pl.ds(off, L)] = z
        cP = plsc.cumsum(t) + prefix
        m = cP >= needP
        lstar = jnp.min(jnp.where(m, iota16, np.int32(64)))
        has = lstar < L
        pj = jnp.max(jnp.where(iota16 == lstar, cP, np.int32(0)))
        cj = jnp.max(jnp.where(iota16 == lstar, t, np.int32(0)))
        first = jnp.logical_and(has, jnp.logical_not(found))
        dstar = jnp.where(first, j * L + lstar, dstar)
        pstar = jnp.where(first, pj, pstar)
        cstar = jnp.where(first, cj, cstar)
        found = jnp.logical_or(found, has)
        prefix = jnp.max(cP)
        return found, dstar, pstar, cstar, prefix

    init = (np.bool_(False), np.int32(0), np.int32(0), np.int32(0),
            np.int32(0))
    _, dstar, pstar, cstar, _ = lax.fori_loop(0, 256 // L, body, init)
    r_new = r - (TOT - pstar)
    return dstar, r_new, cstar


def _body(x_hbm, out_hbm, row_v, cand_v, hist_v):
    wid = lax.axis_index("s") * NC + lax.axis_index("c")
    iota16 = lax.broadcasted_iota(jnp.int32, (L,), 0)
    ones16 = jnp.ones((L,), jnp.int32)
    zeros16f = jnp.zeros((L,), jnp.float32)

    def clr(i, _):
        hist_v[pl.ds(i * L, L)] = jnp.zeros((L,), jnp.int32)
        return 0
    lax.fori_loop(0, 256, clr, 0)

    def per_row(ri, _):
        row = wid * ROWS_PER_W + ri
        pltpu.sync_copy(x_hbm.at[row], row_v)

        def passA(i, _):
            v = row_v[pl.ds(i * L, L)]
            ks = _mono_key(v)
            dg0 = ((_as_u(ks) >> 24)).astype(jnp.int32)
            plsc.addupdate_scatter(hist_v, [iota16 * 256 + dg0], ones16)
            return 0
        lax.fori_loop(0, NV, passA, 0)
        d1, r1, _c1 = _search(hist_v, iota16, np.int32(N), np.int32(K))

        def compact(i, c):
            og, tot1 = c
            v = row_v[pl.ds(i * L, L)]
            ks = _mono_key(v)
            ku = _as_u(ks)
            dg0 = (ku >> 24).astype(jnp.int32)
            meq = dg0 == d1
            plsc.store_compressed(cand_v.at[pl.ds(og, L)], ks, mask=meq)
            dg1 = ((ku >> 16) & np.uint32(255)).astype(jnp.int32)
            plsc.addupdate_scatter(hist_v, [iota16 * 256 + dg1], ones16,
                                   mask=meq)
            npop = jnp.sum(meq.astype(jnp.int32))
            return og + npop, tot1 + npop
        M, tot1 = lax.fori_loop(0, NV, compact, (np.int32(0), np.int32(0)))
        d2, r2, _c2 = _search(hist_v, iota16, tot1, r1)

        nvec = (M + (L - 1)) // L

        def passC(i, tot):
            valid = (i * L + iota16) < M
            ks = cand_v[pl.ds(i * L, L)]
            ku = _as_u(ks)
            dg1 = ((ku >> 16) & np.uint32(255)).astype(jnp.int32)
            meq = jnp.logical_and(valid, dg1 == d2)
            dg2 = ((ku >> 8) & np.uint32(255)).astype(jnp.int32)
            plsc.addupdate_scatter(hist_v, [iota16 * 256 + dg2], ones16,
                                   mask=meq)
            return tot + jnp.sum(meq.astype(jnp.int32))
        tot2 = lax.fori_loop(0, nvec, passC, np.int32(0))
        d3, r3, _c3 = _search(hist_v, iota16, tot2, r2)

        def passD(i, tot):
            valid = (i * L + iota16) < M
            ks = cand_v[pl.ds(i * L, L)]
            ku = _as_u(ks)
            dg1 = ((ku >> 16) & np.uint32(255)).astype(jnp.int32)
            dg2 = ((ku >> 8) & np.uint32(255)).astype(jnp.int32)
            meq = jnp.logical_and(valid,
                                  jnp.logical_and(dg1 == d2, dg2 == d3))
            dg3 = (ku & np.uint32(255)).astype(jnp.int32)
            plsc.addupdate_scatter(hist_v, [iota16 * 256 + dg3], ones16,
                                   mask=meq)
            return tot + jnp.sum(meq.astype(jnp.int32))
        tot3 = lax.fori_loop(0, nvec, passD, np.int32(0))
        d4, r4, eqcnt = _search(hist_v, iota16, tot3, r3)

        kstar = ((d1 << 24) | (d2 << 16) | (d3 << 8) | d4) ^ _I32_MIN
        sstar = kstar

        def final_fast(_):
            def fbody(i, _):
                v = row_v[pl.ds(i * L, L)]
                ss = _mono_key(v) ^ _I32_MIN
                keep = ss >= sstar
                row_v[pl.ds(i * L, L)] = jnp.where(keep, v, zeros16f)
                return 0
            lax.fori_loop(0, NV, fbody, 0)
            return 0

        def final_slow(_):
            def sbody(i, base):
                v = row_v[pl.ds(i * L, L)]
                ss = _mono_key(v) ^ _I32_MIN
                gt = ss > sstar
                eq = ss == sstar
                rank = plsc.cumsum(eq.astype(jnp.int32)) + base
                keep = jnp.logical_or(gt, jnp.logical_and(eq, rank <= r4))
                row_v[pl.ds(i * L, L)] = jnp.where(keep, v, zeros16f)
                return base + jnp.sum(eq.astype(jnp.int32))
            lax.fori_loop(0, NV, sbody, np.int32(0))
            return 0

        lax.cond(r4 == eqcnt, final_fast, final_slow, 0)

        pltpu.sync_copy(row_v, out_hbm.at[row])
        return 0

    lax.fori_loop(0, ROWS_PER_W, per_row, 0)


@jax.jit
def kernel(x):
    mesh = plsc.VectorSubcoreMesh(core_axis_name="c", subcore_axis_name="s",
                                  num_cores=NC, num_subcores=NS)
    f = pl.kernel(
        _body,
        out_type=jax.ShapeDtypeStruct((B, N), jnp.float32),
        mesh=mesh,
        compiler_params=pltpu.CompilerParams(needs_layout_passes=False),
        scratch_types=[
            pltpu.VMEM((N,), jnp.float32),
            pltpu.VMEM((N + L,), jnp.int32),
            pltpu.VMEM((L * 256,), jnp.int32),
        ],
    )
    return f(x)


# speedup vs baseline: 8.6208x; 2.6756x over previous
import functools

import jax
import jax.numpy as jnp
import numpy as np
from jax import lax
from jax.experimental import pallas as pl
from jax.experimental.pallas import tpu as pltpu
from jax.experimental.pallas import tpu_sc as plsc

B = 128
N = 32768
K = 64
L = 16
NC, NS = 2, 16
NW = NC * NS
ROWS_PER_W = B // NW
NV = N // L

_I32_MIN = np.int32(-2147483648)
_M7F = np.int32(0x7FFFFFFF)


def _mono_key(v):
    b = lax.bitcast_convert_type(v, jnp.int32)
    f = b >> 31
    return b ^ (f | _I32_MIN)


def _as_u(k):
    return lax.bitcast_convert_type(k, jnp.uint32)


def _search(hist, iota16, TOT, r):
    needP = TOT - r + 1

    def body(j, c):
        found, dstar, pstar, cstar, prefix = c
        t = jnp.zeros((L,), jnp.int32)
        z = jnp.zeros((L,), jnp.int32)
        for lane in range(L):
            off = lane * 256 + j * L
            t = t + hist[pl.ds(off, L)]
            hist[pl.ds(off, L)] = z
        cP = plsc.cumsum(t) + prefix
        m = cP >= needP
        lstar = jnp.min(jnp.where(m, iota16, np.int32(64)))
        has = lstar < L
        pj = jnp.max(jnp.where(iota16 == lstar, cP, np.int32(0)))
        cj = jnp.max(jnp.where(iota16 == lstar, t, np.int32(0)))
        first = jnp.logical_and(has, jnp.logical_not(found))
        dstar = jnp.where(first, j * L + lstar, dstar)
        pstar = jnp.where(first, pj, pstar)
        cstar = jnp.where(first, cj, cstar)
        found = jnp.logical_or(found, has)
        prefix = jnp.max(cP)
        return found, dstar, pstar, cstar, prefix

    init = (np.bool_(False), np.int32(0), np.int32(0), np.int32(0),
            np.int32(0))
    _, dstar, pstar, cstar, _ = lax.fori_loop(0, 256 // L, body, init)
    r_new = r - (TOT - pstar)
    return dstar, r_new, cstar


def _body(x_hbm, out_hbm, row_v, cand_v, hist_v):
    wid = lax.axis_index("s") * NC + lax.axis_index("c")
    iota16 = lax.broadcasted_iota(jnp.int32, (L,), 0)
    ones16 = jnp.ones((L,), jnp.int32)
    zeros16f = jnp.zeros((L,), jnp.float32)

    @plsc.parallel_loop(0, 256, unroll=8)
    def _(i):
        hist_v[pl.ds(i * L, L)] = jnp.zeros((L,), jnp.int32)

    def per_row(ri, _):
        row = wid * ROWS_PER_W + ri
        pltpu.sync_copy(x_hbm.at[row], row_v)

        @plsc.parallel_loop(0, NV, unroll=8)
        def _(i):
            v = row_v[pl.ds(i * L, L)]
            ks = _mono_key(v)
            dg0 = ((_as_u(ks) >> 24)).astype(jnp.int32)
            plsc.addupdate_scatter(hist_v, [iota16 * 256 + dg0], ones16)
        d1, r1, _c1 = _search(hist_v, iota16, np.int32(N), np.int32(K))

        @plsc.parallel_loop(0, NV, unroll=4, carry=jnp.zeros((), jnp.int32))
        def M(i, og):
            v = row_v[pl.ds(i * L, L)]
            ks = _mono_key(v)
            ku = _as_u(ks)
            dg0 = (ku >> 24).astype(jnp.int32)
            meq = dg0 == d1
            plsc.store_compressed(cand_v.at[pl.ds(og, L)], ks, mask=meq)
            dg1 = ((ku >> 16) & np.uint32(255)).astype(jnp.int32)
            plsc.addupdate_scatter(hist_v, [iota16 * 256 + dg1], ones16,
                                   mask=meq)
            return og + jnp.sum(meq.astype(jnp.int32))
        d2, r2, _c2 = _search(hist_v, iota16, M, r1)

        nvec = (M + (L - 1)) // L

        def passC(i, tot):
            valid = (i * L + iota16) < M
            ks = cand_v[pl.ds(i * L, L)]
            ku = _as_u(ks)
            dg1 = ((ku >> 16) & np.uint32(255)).astype(jnp.int32)
            meq = jnp.logical_and(valid, dg1 == d2)
            dg2 = ((ku >> 8) & np.uint32(255)).astype(jnp.int32)
            plsc.addupdate_scatter(hist_v, [iota16 * 256 + dg2], ones16,
                                   mask=meq)
            return tot + jnp.sum(meq.astype(jnp.int32))
        tot2 = lax.fori_loop(0, nvec, passC, np.int32(0))
        d3, r3, _c3 = _search(hist_v, iota16, tot2, r2)

        def passD(i, tot):
            valid = (i * L + iota16) < M
            ks = cand_v[pl.ds(i * L, L)]
            ku = _as_u(ks)
            dg1 = ((ku >> 16) & np.uint32(255)).astype(jnp.int32)
            dg2 = ((ku >> 8) & np.uint32(255)).astype(jnp.int32)
            meq = jnp.logical_and(valid,
                                  jnp.logical_and(dg1 == d2, dg2 == d3))
            dg3 = (ku & np.uint32(255)).astype(jnp.int32)
            plsc.addupdate_scatter(hist_v, [iota16 * 256 + dg3], ones16,
                                   mask=meq)
            return tot + jnp.sum(meq.astype(jnp.int32))
        tot3 = lax.fori_loop(0, nvec, passD, np.int32(0))
        d4, r4, eqcnt = _search(hist_v, iota16, tot3, r3)

        kstar = ((d1 << 24) | (d2 << 16) | (d3 << 8) | d4) ^ _I32_MIN
        sstar = kstar

        def final_fast(_):
            @plsc.parallel_loop(0, NV, unroll=8)
            def _(i):
                v = row_v[pl.ds(i * L, L)]
                ss = _mono_key(v) ^ _I32_MIN
                keep = ss >= sstar
                row_v[pl.ds(i * L, L)] = jnp.where(keep, v, zeros16f)
            return 0

        def final_slow(_):
            def sbody(i, base):
                v = row_v[pl.ds(i * L, L)]
                ss = _mono_key(v) ^ _I32_MIN
                gt = ss > sstar
                eq = ss == sstar
                rank = plsc.cumsum(eq.astype(jnp.int32)) + base
                keep = jnp.logical_or(gt, jnp.logical_and(eq, rank <= r4))
                row_v[pl.ds(i * L, L)] = jnp.where(keep, v, zeros16f)
                return base + jnp.sum(eq.astype(jnp.int32))
            lax.fori_loop(0, NV, sbody, np.int32(0))
            return 0

        lax.cond(r4 == eqcnt, final_fast, final_slow, 0)

        pltpu.sync_copy(row_v, out_hbm.at[row])
        return 0

    lax.fori_loop(0, ROWS_PER_W, per_row, 0)


@jax.jit
def kernel(x):
    mesh = plsc.VectorSubcoreMesh(core_axis_name="c", subcore_axis_name="s",
                                  num_cores=NC, num_subcores=NS)
    f = pl.kernel(
        _body,
        out_type=jax.ShapeDtypeStruct((B, N), jnp.float32),
        mesh=mesh,
        compiler_params=pltpu.CompilerParams(needs_layout_passes=False),
        scratch_types=[
            pltpu.VMEM((N,), jnp.float32),
            pltpu.VMEM((N + L,), jnp.int32),
            pltpu.VMEM((L * 256,), jnp.int32),
        ],
    )
    return f(x)
